# Initial kernel scaffold; baseline (speedup 1.0000x reference)
import jax
import jax.numpy as jnp
from jax.experimental import pallas as pl


def kernel(x, edge_index, W1, b1, W2, b2):
    raise NotImplementedError("write your pallas kernel here")


# speedup vs baseline: 9.0951x; 9.0951x over previous
import functools

import jax
import jax.numpy as jnp
from jax import lax
from jax.experimental import pallas as pl
from jax.experimental.pallas import tpu as pltpu
from jax.experimental.pallas import tpu_sc as plsc

F32 = jnp.float32

NC = 2
NS = 16
LANE = 128


def _mesh():
    return plsc.VectorSubcoreMesh(
        core_axis_name="c", subcore_axis_name="s", num_cores=NC, num_subcores=NS
    )


def _make_deg(n_pad, e_rows):
    rows_per_tile = e_rows // (NC * NS)
    n_per_tile = n_pad // NS

    @functools.partial(
        pl.kernel,
        out_type=(
            jax.ShapeDtypeStruct((n_pad,), F32),
            jax.ShapeDtypeStruct((n_pad,), F32),
        ),
        mesh=_mesh(),
        scratch_types=[
            pltpu.VMEM_SHARED((n_pad,), F32),
            pltpu.VMEM((rows_per_tile, LANE), jnp.int32),
            pltpu.VMEM((LANE,), F32),
            pltpu.VMEM((n_per_tile,), F32),
        ],
    )
    def deg_kernel(dst2d, zeros1, out0, out1, acc, idx_v, ones_v, wb_v):
        c = lax.axis_index("c")
        s = lax.axis_index("s")
        pltpu.sync_copy(
            zeros1.at[pl.ds(s * n_per_tile, n_per_tile)],
            acc.at[pl.ds(s * n_per_tile, n_per_tile)],
        )
        for i in range(LANE // 16):
            ones_v[pl.ds(i * 16, 16)] = jnp.full((16,), 1.0, F32)
        row0 = c * (e_rows // NC) + s * rows_per_tile
        pltpu.sync_copy(dst2d.at[pl.ds(row0, rows_per_tile)], idx_v)
        plsc.subcore_barrier()

        def body(j, _):
            pltpu.sync_copy(ones_v, acc.at[idx_v.at[j]], add=True)
            return 0

        lax.fori_loop(0, rows_per_tile, body, 0)
        plsc.subcore_barrier()
        sl = pl.ds(s * n_per_tile, n_per_tile)
        pltpu.sync_copy(acc.at[sl], wb_v)

        @pl.when(c == 0)
        def _():
            pltpu.sync_copy(wb_v, out0.at[sl])

        @pl.when(c == 1)
        def _():
            pltpu.sync_copy(wb_v, out1.at[sl])

    return deg_kernel


def _make_agg(n_pad, e_rows, half):
    rows_per_tile = e_rows // NS
    n_per_tile = n_pad // NS
    wb_chunks = n_per_tile // LANE

    @functools.partial(
        pl.kernel,
        out_type=(
            jax.ShapeDtypeStruct((n_pad, half), F32),
            jax.ShapeDtypeStruct((n_pad, half), F32),
        ),
        mesh=_mesh(),
        scratch_types=[
            pltpu.VMEM_SHARED((n_pad, half), F32),
            pltpu.VMEM((rows_per_tile, LANE), jnp.int32),
            pltpu.VMEM((rows_per_tile, LANE), jnp.int32),
            pltpu.VMEM((LANE, half), F32),
            pltpu.SemaphoreType.DMA,
        ],
    )
    def agg_kernel(src2d, dst2d, tab_lo, tab_hi, zeros2,
                   out_lo, out_hi, acc, sidx, didx, rows_v, sem):
        c = lax.axis_index("c")
        s = lax.axis_index("s")
        nsl = pl.ds(s * n_per_tile, n_per_tile)
        pltpu.sync_copy(zeros2.at[nsl], acc.at[nsl])
        pltpu.sync_copy(src2d.at[pl.ds(s * rows_per_tile, rows_per_tile)], sidx)
        pltpu.sync_copy(dst2d.at[pl.ds(s * rows_per_tile, rows_per_tile)], didx)
        plsc.subcore_barrier()

        def run(tab, out):
            def body(j, _):
                pltpu.async_copy(tab.at[sidx.at[j]], rows_v, sem).wait()
                pltpu.sync_copy(rows_v, acc.at[didx.at[j]], add=True)
                return 0

            lax.fori_loop(0, rows_per_tile, body, 0)
            plsc.subcore_barrier()
            for q in range(wb_chunks):
                sl = pl.ds(s * n_per_tile + q * LANE, LANE)
                pltpu.sync_copy(acc.at[sl], rows_v)
                pltpu.sync_copy(rows_v, out.at[sl])

        @pl.when(c == 0)
        def _():
            run(tab_lo, out_lo)

        @pl.when(c == 1)
        def _():
            run(tab_hi, out_hi)

    return agg_kernel


def _tc_scale(deg0, deg1, x_pad, half):
    n_pad, fin = x_pad.shape
    blk = 1024
    grid = (n_pad // blk,)

    def body(d0, d1, x, lo, hi):
        dinv = lax.rsqrt(d0[...] + d1[...] + 1.0)
        xs = x[...] * dinv[:, None]
        lo[...] = xs[:, :half]
        hi[...] = xs[:, half:]

    return pl.pallas_call(
        body,
        grid=grid,
        in_specs=[
            pl.BlockSpec((blk,), lambda i: (i,)),
            pl.BlockSpec((blk,), lambda i: (i,)),
            pl.BlockSpec((blk, fin), lambda i: (i, 0)),
        ],
        out_specs=[
            pl.BlockSpec((blk, half), lambda i: (i, 0)),
            pl.BlockSpec((blk, half), lambda i: (i, 0)),
        ],
        out_shape=[
            jax.ShapeDtypeStruct((n_pad, half), F32),
            jax.ShapeDtypeStruct((n_pad, half), F32),
        ],
    )(deg0, deg1, x_pad)


def _tc_layer1(deg0, deg1, x_pad, u_lo, u_hi, W1, b1, W2, half):
    n_pad, fin = x_pad.shape
    fmid = W1.shape[1]
    blk = 1024
    grid = (n_pad // blk,)

    def body(d0, d1, x, ulo, uhi, w1, bb1, w2, t_out, tslo, tshi):
        dinv = lax.rsqrt(d0[...] + d1[...] + 1.0)
        u = jnp.concatenate([ulo[...], uhi[...]], axis=1)
        z = u * dinv[:, None] + x[...] * (dinv * dinv)[:, None]
        h = jnp.maximum(
            jnp.dot(z, w1[...], preferred_element_type=F32) + bb1[...][None, :],
            0.0,
        )
        t = jnp.dot(h, w2[...], preferred_element_type=F32)
        t_out[...] = t
        ts = t * dinv[:, None]
        tslo[...] = ts[:, :half]
        tshi[...] = ts[:, half:]

    return pl.pallas_call(
        body,
        grid=grid,
        in_specs=[
            pl.BlockSpec((blk,), lambda i: (i,)),
            pl.BlockSpec((blk,), lambda i: (i,)),
            pl.BlockSpec((blk, fin), lambda i: (i, 0)),
            pl.BlockSpec((blk, half), lambda i: (i, 0)),
            pl.BlockSpec((blk, half), lambda i: (i, 0)),
            pl.BlockSpec((fin, fmid), lambda i: (0, 0)),
            pl.BlockSpec((fmid,), lambda i: (0,)),
            pl.BlockSpec((fmid, fin), lambda i: (0, 0)),
        ],
        out_specs=[
            pl.BlockSpec((blk, fin), lambda i: (i, 0)),
            pl.BlockSpec((blk, half), lambda i: (i, 0)),
            pl.BlockSpec((blk, half), lambda i: (i, 0)),
        ],
        out_shape=[
            jax.ShapeDtypeStruct((n_pad, fin), F32),
            jax.ShapeDtypeStruct((n_pad, half), F32),
            jax.ShapeDtypeStruct((n_pad, half), F32),
        ],
    )(deg0, deg1, x_pad, u_lo, u_hi, W1, b1, W2)


def _tc_finish(deg0, deg1, t, u_lo, u_hi, b2):
    n_pad, fout = t.shape
    half = fout // 2
    blk = 1024
    grid = (n_pad // blk,)

    def body(d0, d1, tt, ulo, uhi, bb2, out):
        dinv = lax.rsqrt(d0[...] + d1[...] + 1.0)
        u = jnp.concatenate([ulo[...], uhi[...]], axis=1)
        z = u * dinv[:, None] + tt[...] * (dinv * dinv)[:, None] + bb2[...][None, :]
        r = jnp.maximum(z, 0.0)
        m = jnp.max(r, axis=1, keepdims=True)
        lse = m + jnp.log(jnp.sum(jnp.exp(r - m), axis=1, keepdims=True))
        out[...] = r - lse

    return pl.pallas_call(
        body,
        grid=grid,
        in_specs=[
            pl.BlockSpec((blk,), lambda i: (i,)),
            pl.BlockSpec((blk,), lambda i: (i,)),
            pl.BlockSpec((blk, fout), lambda i: (i, 0)),
            pl.BlockSpec((blk, half), lambda i: (i, 0)),
            pl.BlockSpec((blk, half), lambda i: (i, 0)),
            pl.BlockSpec((fout,), lambda i: (0,)),
        ],
        out_specs=pl.BlockSpec((blk, fout), lambda i: (i, 0)),
        out_shape=jax.ShapeDtypeStruct((n_pad, fout), F32),
    )(deg0, deg1, t, u_lo, u_hi, b2)


def kernel(x, edge_index, W1, b1, W2, b2):
    n, fin = x.shape
    half = fin // 2
    e = edge_index.shape[1]

    n_pad = ((n + 1 + 1023) // 1024) * 1024
    e_pad = ((e + NC * NS * LANE - 1) // (NC * NS * LANE)) * (NC * NS * LANE)

    ei = edge_index.astype(jnp.int32)
    pad = jnp.full((e_pad - e,), n, jnp.int32)
    src2d = jnp.concatenate([ei[0], pad]).reshape(e_pad // LANE, LANE)
    dst2d = jnp.concatenate([ei[1], pad]).reshape(e_pad // LANE, LANE)
    x_pad = jnp.pad(x, ((0, n_pad - n), (0, 0)))
    zeros1 = jnp.zeros((n_pad,), F32)
    zeros2 = jnp.zeros((n_pad, half), F32)

    e_rows = e_pad // LANE
    deg0, deg1 = _make_deg(n_pad, e_rows)(dst2d, zeros1)
    xs_lo, xs_hi = _tc_scale(deg0, deg1, x_pad, half)
    agg = _make_agg(n_pad, e_rows, half)
    u1_lo, u1_hi = agg(src2d, dst2d, xs_lo, xs_hi, zeros2)
    t, ts_lo, ts_hi = _tc_layer1(deg0, deg1, x_pad, u1_lo, u1_hi, W1, b1, W2, half)
    u2_lo, u2_hi = agg(src2d, dst2d, ts_lo, ts_hi, zeros2)
    o = _tc_finish(deg0, deg1, t, u2_lo, u2_hi, b2)
    return o[:n]


# speedup vs baseline: 10.7217x; 1.1788x over previous
import functools

import jax
import jax.numpy as jnp
from jax import lax
from jax.experimental import pallas as pl
from jax.experimental.pallas import tpu as pltpu
from jax.experimental.pallas import tpu_sc as plsc

F32 = jnp.float32

NC = 2
NS = 16
LANE = 128


def _mesh():
    return plsc.VectorSubcoreMesh(
        core_axis_name="c", subcore_axis_name="s", num_cores=NC, num_subcores=NS
    )


def _make_deg(n_pad, e_rows):
    rows_per_tile = e_rows // (NC * NS)
    n_per_tile = n_pad // NS

    @functools.partial(
        pl.kernel,
        out_type=(
            jax.ShapeDtypeStruct((n_pad,), F32),
            jax.ShapeDtypeStruct((n_pad,), F32),
        ),
        mesh=_mesh(),
        scratch_types=[
            pltpu.VMEM_SHARED((n_pad,), F32),
            pltpu.VMEM((rows_per_tile, LANE), jnp.int32),
            pltpu.VMEM((LANE,), F32),
            pltpu.VMEM((n_per_tile,), F32),
        ],
    )
    def deg_kernel(dst2d, zeros1, out0, out1, acc, idx_v, ones_v, wb_v):
        c = lax.axis_index("c")
        s = lax.axis_index("s")
        pltpu.sync_copy(
            zeros1.at[pl.ds(s * n_per_tile, n_per_tile)],
            acc.at[pl.ds(s * n_per_tile, n_per_tile)],
        )
        for i in range(LANE // 16):
            ones_v[pl.ds(i * 16, 16)] = jnp.full((16,), 1.0, F32)
        row0 = c * (e_rows // NC) + s * rows_per_tile
        pltpu.sync_copy(dst2d.at[pl.ds(row0, rows_per_tile)], idx_v)
        plsc.subcore_barrier()

        def body(j, _):
            pltpu.sync_copy(ones_v, acc.at[idx_v.at[j]], add=True)
            return 0

        lax.fori_loop(0, rows_per_tile, body, 0)
        plsc.subcore_barrier()
        sl = pl.ds(s * n_per_tile, n_per_tile)
        pltpu.sync_copy(acc.at[sl], wb_v)

        @pl.when(c == 0)
        def _():
            pltpu.sync_copy(wb_v, out0.at[sl])

        @pl.when(c == 1)
        def _():
            pltpu.sync_copy(wb_v, out1.at[sl])

    return deg_kernel


def _make_agg(n_pad, e_rows, half):
    rows_per_tile = e_rows // NS
    n_per_tile = n_pad // NS
    wb_chunks = n_per_tile // LANE

    nib = 4
    ngb = 2
    assert rows_per_tile % nib == 0

    @functools.partial(
        pl.kernel,
        out_type=(
            jax.ShapeDtypeStruct((n_pad, half), F32),
            jax.ShapeDtypeStruct((n_pad, half), F32),
        ),
        mesh=_mesh(),
        scratch_types=[
            pltpu.VMEM_SHARED((n_pad, half), F32),
            [pltpu.VMEM((2, LANE), jnp.int32) for _ in range(nib)],
            [pltpu.VMEM((LANE, half), F32) for _ in range(ngb)],
            [pltpu.SemaphoreType.DMA for _ in range(nib)],
            [pltpu.SemaphoreType.DMA for _ in range(ngb)],
        ],
    )
    def agg_kernel(edg3d, tab_lo, tab_hi, zeros2,
                   out_lo, out_hi, acc, idx_v, rows_v, isems, gsems):
        c = lax.axis_index("c")
        s = lax.axis_index("s")
        nsl = pl.ds(s * n_per_tile, n_per_tile)
        pltpu.sync_copy(zeros2.at[nsl], acc.at[nsl])
        plsc.subcore_barrier()
        row0 = s * rows_per_tile

        def run(tab, out):
            def prefetch(j, ib):
                pltpu.async_copy(edg3d.at[row0 + j], idx_v[ib], isems[ib])

            def wait_idx(ib):
                pltpu.make_async_copy(edg3d.at[row0], idx_v[ib],
                                      isems[ib]).wait()

            def gather(ib, gb):
                pltpu.async_copy(tab.at[idx_v[ib].at[0]], rows_v[gb],
                                 gsems[gb])

            def wait_gather(gb):
                pltpu.make_async_copy(tab.at[idx_v[0].at[0]], rows_v[gb],
                                      gsems[gb]).wait()

            for j in range(nib):
                prefetch(j, j)
            for j in range(ngb):
                wait_idx(j)
                gather(j, j)

            def outer(i, _):
                for b in range(nib):
                    j = i * nib + b
                    gb = b % ngb
                    ib2 = (b + ngb) % nib
                    wait_gather(gb)
                    pltpu.sync_copy(rows_v[gb], acc.at[idx_v[b].at[1]],
                                    add=True)
                    pl.when(j + nib < rows_per_tile)(
                        lambda j=j, b=b: prefetch(j + nib, b))

                    def nxt_gather(ib2=ib2, gb=gb):
                        wait_idx(ib2)
                        gather(ib2, gb)

                    pl.when(j + ngb < rows_per_tile)(nxt_gather)
                return 0

            lax.fori_loop(0, rows_per_tile // nib, outer, 0)
            plsc.subcore_barrier()
            for q in range(wb_chunks):
                sl = pl.ds(s * n_per_tile + q * LANE, LANE)
                pltpu.sync_copy(acc.at[sl], rows_v[0])
                pltpu.sync_copy(rows_v[0], out.at[sl])

        @pl.when(c == 0)
        def _():
            run(tab_lo, out_lo)

        @pl.when(c == 1)
        def _():
            run(tab_hi, out_hi)

    return agg_kernel


def _tc_scale(deg0, deg1, x_pad, half):
    n_pad, fin = x_pad.shape
    blk = 1024
    grid = (n_pad // blk,)

    def body(d0, d1, x, lo, hi):
        dinv = lax.rsqrt(d0[...] + d1[...] + 1.0)
        xs = x[...] * dinv[:, None]
        lo[...] = xs[:, :half]
        hi[...] = xs[:, half:]

    return pl.pallas_call(
        body,
        grid=grid,
        in_specs=[
            pl.BlockSpec((blk,), lambda i: (i,)),
            pl.BlockSpec((blk,), lambda i: (i,)),
            pl.BlockSpec((blk, fin), lambda i: (i, 0)),
        ],
        out_specs=[
            pl.BlockSpec((blk, half), lambda i: (i, 0)),
            pl.BlockSpec((blk, half), lambda i: (i, 0)),
        ],
        out_shape=[
            jax.ShapeDtypeStruct((n_pad, half), F32),
            jax.ShapeDtypeStruct((n_pad, half), F32),
        ],
    )(deg0, deg1, x_pad)


def _tc_layer1(deg0, deg1, x_pad, u_lo, u_hi, W1, b1, W2, half):
    n_pad, fin = x_pad.shape
    fmid = W1.shape[1]
    blk = 1024
    grid = (n_pad // blk,)

    def body(d0, d1, x, ulo, uhi, w1, bb1, w2, t_out, tslo, tshi):
        dinv = lax.rsqrt(d0[...] + d1[...] + 1.0)
        u = jnp.concatenate([ulo[...], uhi[...]], axis=1)
        z = u * dinv[:, None] + x[...] * (dinv * dinv)[:, None]
        h = jnp.maximum(
            jnp.dot(z, w1[...], preferred_element_type=F32) + bb1[...][None, :],
            0.0,
        )
        t = jnp.dot(h, w2[...], preferred_element_type=F32)
        t_out[...] = t
        ts = t * dinv[:, None]
        tslo[...] = ts[:, :half]
        tshi[...] = ts[:, half:]

    return pl.pallas_call(
        body,
        grid=grid,
        in_specs=[
            pl.BlockSpec((blk,), lambda i: (i,)),
            pl.BlockSpec((blk,), lambda i: (i,)),
            pl.BlockSpec((blk, fin), lambda i: (i, 0)),
            pl.BlockSpec((blk, half), lambda i: (i, 0)),
            pl.BlockSpec((blk, half), lambda i: (i, 0)),
            pl.BlockSpec((fin, fmid), lambda i: (0, 0)),
            pl.BlockSpec((fmid,), lambda i: (0,)),
            pl.BlockSpec((fmid, fin), lambda i: (0, 0)),
        ],
        out_specs=[
            pl.BlockSpec((blk, fin), lambda i: (i, 0)),
            pl.BlockSpec((blk, half), lambda i: (i, 0)),
            pl.BlockSpec((blk, half), lambda i: (i, 0)),
        ],
        out_shape=[
            jax.ShapeDtypeStruct((n_pad, fin), F32),
            jax.ShapeDtypeStruct((n_pad, half), F32),
            jax.ShapeDtypeStruct((n_pad, half), F32),
        ],
    )(deg0, deg1, x_pad, u_lo, u_hi, W1, b1, W2)


def _tc_finish(deg0, deg1, t, u_lo, u_hi, b2):
    n_pad, fout = t.shape
    half = fout // 2
    blk = 1024
    grid = (n_pad // blk,)

    def body(d0, d1, tt, ulo, uhi, bb2, out):
        dinv = lax.rsqrt(d0[...] + d1[...] + 1.0)
        u = jnp.concatenate([ulo[...], uhi[...]], axis=1)
        z = u * dinv[:, None] + tt[...] * (dinv * dinv)[:, None] + bb2[...][None, :]
        r = jnp.maximum(z, 0.0)
        m = jnp.max(r, axis=1, keepdims=True)
        lse = m + jnp.log(jnp.sum(jnp.exp(r - m), axis=1, keepdims=True))
        out[...] = r - lse

    return pl.pallas_call(
        body,
        grid=grid,
        in_specs=[
            pl.BlockSpec((blk,), lambda i: (i,)),
            pl.BlockSpec((blk,), lambda i: (i,)),
            pl.BlockSpec((blk, fout), lambda i: (i, 0)),
            pl.BlockSpec((blk, half), lambda i: (i, 0)),
            pl.BlockSpec((blk, half), lambda i: (i, 0)),
            pl.BlockSpec((fout,), lambda i: (0,)),
        ],
        out_specs=pl.BlockSpec((blk, fout), lambda i: (i, 0)),
        out_shape=jax.ShapeDtypeStruct((n_pad, fout), F32),
    )(deg0, deg1, t, u_lo, u_hi, b2)


def kernel(x, edge_index, W1, b1, W2, b2):
    n, fin = x.shape
    half = fin // 2
    e = edge_index.shape[1]

    n_pad = ((n + 1 + 1023) // 1024) * 1024
    e_pad = ((e + NC * NS * LANE - 1) // (NC * NS * LANE)) * (NC * NS * LANE)

    ei = edge_index.astype(jnp.int32)
    pad = jnp.full((e_pad - e,), n, jnp.int32)
    src2d = jnp.concatenate([ei[0], pad]).reshape(e_pad // LANE, LANE)
    dst2d = jnp.concatenate([ei[1], pad]).reshape(e_pad // LANE, LANE)
    edg3d = jnp.stack([src2d, dst2d], axis=1)
    x_pad = jnp.pad(x, ((0, n_pad - n), (0, 0)))
    zeros1 = jnp.zeros((n_pad,), F32)
    zeros2 = jnp.zeros((n_pad, half), F32)

    e_rows = e_pad // LANE
    deg0, deg1 = _make_deg(n_pad, e_rows)(dst2d, zeros1)
    xs_lo, xs_hi = _tc_scale(deg0, deg1, x_pad, half)
    agg = _make_agg(n_pad, e_rows, half)
    u1_lo, u1_hi = agg(edg3d, xs_lo, xs_hi, zeros2)
    t, ts_lo, ts_hi = _tc_layer1(deg0, deg1, x_pad, u1_lo, u1_hi, W1, b1, W2, half)
    u2_lo, u2_hi = agg(edg3d, ts_lo, ts_hi, zeros2)
    o = _tc_finish(deg0, deg1, t, u2_lo, u2_hi, b2)
    return o[:n]
